# baseline (speedup 1.0000x reference)
import functools

import jax
import jax.numpy as jnp
from jax import lax
from jax.experimental import pallas as pl
from jax.experimental.pallas import tpu as pltpu
from jax.experimental.pallas import tpu_sc as plsc

N_NODES = 10000
N_EDGES = 320000
D = 128
N_GRAPHS = 64

NW = 32
EPT = 10240
CH = 80
NBUF = 4
CPT = EPT // CH
NG = CPT // NBUF
PAD_N = 10240
RPS = PAD_N // 16
NB = 1000


def _mm_body(x_ref, w_ref, o_ref):
    o_ref[...] = jnp.dot(x_ref[...], w_ref[...],
                         preferred_element_type=jnp.float32)


def _project(x, w, blk):
    n, k = x.shape
    m = w.shape[1]
    return pl.pallas_call(
        _mm_body,
        grid=(n // blk,),
        in_specs=[
            pl.BlockSpec((blk, k), lambda i: (i, 0)),
            pl.BlockSpec((k, m), lambda i: (0, 0)),
        ],
        out_specs=pl.BlockSpec((blk, m), lambda i: (i, 0)),
        out_shape=jax.ShapeDtypeStruct((n, m), jnp.float32),
    )(x, w)


def _edge_agg_body(y_hbm, ew_hbm, src_hbm, dst_hbm, out_hbm,
                   bufs, sidx, didx, agg, sem_l, sem_g, sem_s):
    cid = lax.axis_index("c")
    sid = lax.axis_index("s")
    wid = cid * 16 + sid

    def base_of(c):
        return pl.multiple_of(wid * EPT + c * CH, 8)

    def start_loads(c, j):
        base = base_of(c)
        pltpu.async_copy(ew_hbm.at[pl.ds(base, CH)], bufs.at[j], sem_l.at[j])
        pltpu.async_copy(src_hbm.at[pl.ds(base, CH)], sidx.at[j],
                         sem_l.at[j])
        pltpu.async_copy(dst_hbm.at[pl.ds(base, CH)], didx.at[j],
                         sem_l.at[j])

    def wait_loads(c, j):
        base = base_of(c)
        pltpu.make_async_copy(ew_hbm.at[pl.ds(base, CH)], bufs.at[j],
                              sem_l.at[j]).wait()
        pltpu.make_async_copy(src_hbm.at[pl.ds(base, CH)], sidx.at[j],
                              sem_l.at[j]).wait()
        pltpu.make_async_copy(dst_hbm.at[pl.ds(base, CH)], didx.at[j],
                              sem_l.at[j]).wait()

    zv = jnp.zeros((16,), jnp.float32)

    @plsc.parallel_loop(0, CH, 1, unroll=4)
    def _(r):
        for gg in range(D // 16):
            bufs[0, r, pl.ds(gg * 16, 16)] = zv

    for t in range(RPS // CH):
        pltpu.sync_copy(bufs.at[0],
                        agg.at[pl.ds(sid * RPS + t * CH, CH)])

    for j in range(NBUF):
        start_loads(j, j)
    plsc.subcore_barrier()

    last_real = (N_EDGES - (NW - 1) * EPT) // CH // NBUF
    ngroups = jnp.where(wid == NW - 1, last_real, NG)

    def group(g, carry):
        c0 = g * NBUF
        gather_hs = []
        for j in range(NBUF):
            wait_loads(c0 + j, j)
            h = pltpu.async_copy(y_hbm.at[sidx.at[j]], bufs.at[j],
                                 sem_g.at[j], add=True)
            gather_hs.append(h)
        scatter_hs = []
        for j in range(NBUF):
            gather_hs[j].wait()

            @plsc.parallel_loop(0, CH, 1, unroll=4)
            def _(r, j=j):
                for gg in range(D // 16):
                    s = pl.ds(gg * 16, 16)
                    bufs[j, r, s] = jnp.maximum(bufs[j, r, s], 0.0)

            h = pltpu.async_copy(bufs.at[j], agg.at[didx.at[j]],
                                 sem_s.at[j], add=True)
            scatter_hs.append(h)
        for j in range(NBUF):
            scatter_hs[j].wait()

            @pl.when(g + 1 < ngroups)
            def _(j=j):
                start_loads((g + 1) * NBUF + j, j)
        return carry

    lax.fori_loop(0, ngroups, group, 0)
    plsc.subcore_barrier()
    pltpu.sync_copy(agg.at[pl.ds(sid * RPS, RPS)],
                    out_hbm.at[pl.ds(cid * PAD_N + sid * RPS, RPS)])


@functools.cache
def _edge_agg_kernel():
    return pl.kernel(
        _edge_agg_body,
        mesh=plsc.VectorSubcoreMesh(core_axis_name="c", subcore_axis_name="s"),
        out_type=jax.ShapeDtypeStruct((2 * PAD_N, D), jnp.float32),
        scratch_types=[
            pltpu.VMEM((NBUF, CH, D), jnp.float32),
            pltpu.VMEM((NBUF, CH), jnp.int32),
            pltpu.VMEM((NBUF, CH), jnp.int32),
            pltpu.VMEM_SHARED((PAD_N, D), jnp.float32),
            pltpu.SemaphoreType.DMA((NBUF,)),
            pltpu.SemaphoreType.DMA((NBUF,)),
            pltpu.SemaphoreType.DMA((NBUF,)),
        ],
    )


def _edge_agg(*args):
    return _edge_agg_kernel()(*args)


def _pool_body(x_ref, a0_ref, a1_ref, gid_ref, w_ref, b_ref, o_ref,
               sums, counts):
    i = pl.program_id(0)

    @pl.when(i == 0)
    def _():
        sums[...] = jnp.zeros_like(sums)
        counts[...] = jnp.zeros_like(counts)

    nh = jnp.maximum(
        jnp.dot(x_ref[...], w_ref[...], preferred_element_type=jnp.float32)
        + b_ref[...] + a0_ref[0] + a1_ref[0], 0.0)
    ids = jnp.broadcast_to(gid_ref[0], (N_GRAPHS, NB))
    iota = lax.broadcasted_iota(jnp.int32, (N_GRAPHS, NB), 0)
    onehot = (iota == ids).astype(jnp.float32)
    sums[...] += jnp.dot(onehot, nh, preferred_element_type=jnp.float32)
    counts[...] += jnp.sum(onehot, axis=1, keepdims=True)

    @pl.when(i == pl.num_programs(0) - 1)
    def _():
        o_ref[...] = jnp.maximum(
            sums[...] / jnp.maximum(counts[...], 1.0), 0.0)


def _pool(x, aggs3, gid3, w_self, b2):
    return pl.pallas_call(
        _pool_body,
        grid=(N_NODES // NB,),
        in_specs=[
            pl.BlockSpec((NB, D), lambda i: (i, 0)),
            pl.BlockSpec((1, NB, D), lambda i: (0, i, 0)),
            pl.BlockSpec((1, NB, D), lambda i: (1, i, 0)),
            pl.BlockSpec((1, 1, NB), lambda i: (i, 0, 0)),
            pl.BlockSpec((D, D), lambda i: (0, 0)),
            pl.BlockSpec((1, D), lambda i: (0, 0)),
        ],
        out_specs=pl.BlockSpec((N_GRAPHS, D), lambda i: (0, 0)),
        out_shape=jax.ShapeDtypeStruct((N_GRAPHS, D), jnp.float32),
        scratch_shapes=[
            pltpu.VMEM((N_GRAPHS, D), jnp.float32),
            pltpu.VMEM((N_GRAPHS, 1), jnp.float32),
        ],
        compiler_params=pltpu.CompilerParams(
            dimension_semantics=("arbitrary",)),
    )(x, aggs3, aggs3, gid3, w_self, b2)


def kernel(x, edge_index, edge_attr, graph_ids, W_self, W_src, W_e, b):
    src = edge_index[0].astype(jnp.int32)
    dst = edge_index[1].astype(jnp.int32)

    y = _project(x, W_src, NB)
    ew = _project(edge_attr, W_e, 4000)

    aggs = _edge_agg(y, ew, src, dst)
    aggs3 = aggs.reshape(2, PAD_N, D)

    gid3 = graph_ids.astype(jnp.int32).reshape(N_NODES // NB, 1, NB)
    return _pool(x, aggs3, gid3, W_self, b.reshape(1, D))
